# baseline (speedup 1.0000x reference)
import functools

import jax
import jax.numpy as jnp
from jax import lax
from jax.experimental import pallas as pl
from jax.experimental.pallas import tpu as pltpu
from jax.experimental.pallas import tpu_sc as plsc

NUM_EMBEDDINGS = 1000000
EMBED_DIM = 64
PADDING_IDX = 0
BATCH = 4096
SEQ = 200

NC = 2
NS = 16
NW = NC * NS
N = BATCH * SEQ
PER_W = N // NW
CHUNK = 128
NCHUNK = PER_W // CHUNK

_MESH = plsc.VectorSubcoreMesh(
    core_axis_name="c", subcore_axis_name="s", num_cores=NC, num_subcores=NS
)


@functools.partial(
    pl.kernel,
    out_type=jax.ShapeDtypeStruct((N, EMBED_DIM), jnp.float32),
    mesh=_MESH,
    scratch_types=[
        pltpu.VMEM((NCHUNK, CHUNK), jnp.int32),
        pltpu.VMEM((CHUNK, EMBED_DIM), jnp.float32),
        pltpu.VMEM((CHUNK, EMBED_DIM), jnp.float32),
        pltpu.SemaphoreType.DMA,
        pltpu.SemaphoreType.DMA,
    ],
    compiler_params=pltpu.CompilerParams(use_tc_tiling_on_sc=False),
)
def _gather_sc(idx_hbm, table_hbm, out_hbm, idx_v, rows0, rows1, sem0, sem1):
    wid = lax.axis_index("s") * NC + lax.axis_index("c")
    base = wid * PER_W
    pltpu.sync_copy(idx_hbm.at[wid], idx_v)

    bufs = (rows0, rows1)
    sems = (sem0, sem1)

    def start(i, b):
        pltpu.async_copy(table_hbm.at[idx_v.at[i]], bufs[b], sems[b])

    def wait(i, b):
        pltpu.make_async_copy(table_hbm.at[idx_v.at[i]], bufs[b], sems[b]).wait()

    start(0, 0)

    def outer(k, carry):
        i0 = k * 2
        for b in range(2):
            i = i0 + b

            @pl.when(i + 1 < NCHUNK)
            def _():
                start(i + 1, 1 - b)

            wait(i, b)
            pltpu.sync_copy(bufs[b], out_hbm.at[pl.ds(base + i * CHUNK, CHUNK)])
        return carry

    lax.fori_loop(0, NCHUNK // 2, outer, 0)


def _mask_body(x_ref, m_ref):
    m_ref[...] = (x_ref[...] != PADDING_IDX).astype(jnp.float32)


def _mask_tc(x):
    return pl.pallas_call(
        _mask_body,
        out_shape=jax.ShapeDtypeStruct((BATCH, SEQ), jnp.float32),
        grid=(8,),
        in_specs=[pl.BlockSpec((BATCH // 8, SEQ), lambda i: (i, 0))],
        out_specs=pl.BlockSpec((BATCH // 8, SEQ), lambda i: (i, 0)),
    )(x)


def kernel(x, table):
    idx = x.reshape(NW, NCHUNK, CHUNK)
    out = _gather_sc(idx, table)
    mask = _mask_tc(x)
    return out.reshape(BATCH, SEQ, EMBED_DIM), mask


# speedup vs baseline: 1.0166x; 1.0166x over previous
import functools

import jax
import jax.numpy as jnp
from jax import lax
from jax.experimental import pallas as pl
from jax.experimental.pallas import tpu as pltpu
from jax.experimental.pallas import tpu_sc as plsc

NUM_EMBEDDINGS = 1000000
EMBED_DIM = 64
PADDING_IDX = 0
BATCH = 4096
SEQ = 200

NC = 2
NS = 16
NW = NC * NS
N = BATCH * SEQ
PER_W = N // NW
CHUNK = 128
NCHUNK = PER_W // CHUNK

NBUF = 8
GLEAD = 5

_MESH = plsc.VectorSubcoreMesh(
    core_axis_name="c", subcore_axis_name="s", num_cores=NC, num_subcores=NS
)


@functools.partial(
    pl.kernel,
    out_type=jax.ShapeDtypeStruct((N, EMBED_DIM), jnp.float32),
    mesh=_MESH,
    scratch_types=[
        pltpu.VMEM((NCHUNK, CHUNK), jnp.int32),
        pltpu.VMEM((NBUF, CHUNK, EMBED_DIM), jnp.float32),
        pltpu.SemaphoreType.DMA((NBUF,)),
        pltpu.SemaphoreType.DMA((NBUF,)),
    ],
    compiler_params=pltpu.CompilerParams(use_tc_tiling_on_sc=False),
)
def _gather_sc(idx_hbm, table_hbm, out_hbm, idx_v, rows_v, gsem, ssem):
    wid = lax.axis_index("s") * NC + lax.axis_index("c")
    base = wid * PER_W
    pltpu.sync_copy(idx_hbm.at[wid], idx_v)

    def g_desc(i, b):
        return pltpu.make_async_copy(
            table_hbm.at[idx_v.at[i]], rows_v.at[b], gsem.at[b]
        )

    def s_desc(i, b):
        return pltpu.make_async_copy(
            rows_v.at[b],
            out_hbm.at[pl.ds(base + i * CHUNK, CHUNK)],
            ssem.at[b],
        )

    for b in range(GLEAD + 1):
        g_desc(b, b).start()

    def outer(k, carry):
        i0 = k * NBUF
        for b in range(NBUF):
            i = i0 + b
            g_desc(i, b).wait()
            s_desc(i, b).start()
            j = i + GLEAD + 1
            bj = (b + GLEAD + 1) % NBUF

            @pl.when(jnp.logical_and(j >= NBUF, j < NCHUNK))
            def _():
                s_desc(j - NBUF, bj).wait()

            @pl.when(j < NCHUNK)
            def _():
                g_desc(j, bj).start()
        return carry

    lax.fori_loop(0, NCHUNK // NBUF, outer, 0)

    for b in range(NBUF):
        i = NCHUNK - NBUF + b
        s_desc(i, i % NBUF).wait()


def _mask_body(x_ref, m_ref):
    m_ref[...] = (x_ref[...] != PADDING_IDX).astype(jnp.float32)


def _mask_tc(x):
    return pl.pallas_call(
        _mask_body,
        out_shape=jax.ShapeDtypeStruct((BATCH, SEQ), jnp.float32),
        grid=(8,),
        in_specs=[pl.BlockSpec((BATCH // 8, SEQ), lambda i: (i, 0))],
        out_specs=pl.BlockSpec((BATCH // 8, SEQ), lambda i: (i, 0)),
    )(x)


def kernel(x, table):
    idx = x.reshape(NW, NCHUNK, CHUNK)
    out = _gather_sc(idx, table)
    mask = _mask_tc(x)
    return out.reshape(BATCH, SEQ, EMBED_DIM), mask


# speedup vs baseline: 1.0201x; 1.0034x over previous
import functools

import jax
import jax.numpy as jnp
from jax import lax
from jax.experimental import pallas as pl
from jax.experimental.pallas import tpu as pltpu
from jax.experimental.pallas import tpu_sc as plsc

NUM_EMBEDDINGS = 1000000
EMBED_DIM = 64
PADDING_IDX = 0
BATCH = 4096
SEQ = 200

NC = 2
NS = 16
NW = NC * NS
BPW = BATCH // NW
SPLIT = 104
NCH = BPW * 2

NBUF = 8
GLEAD = 5

_MESH = plsc.VectorSubcoreMesh(
    core_axis_name="c", subcore_axis_name="s", num_cores=NC, num_subcores=NS
)


@functools.partial(
    pl.kernel,
    out_type=jax.ShapeDtypeStruct((BATCH, SEQ, EMBED_DIM), jnp.float32),
    mesh=_MESH,
    scratch_types=[
        pltpu.VMEM((BPW, SEQ), jnp.int32),
        pltpu.VMEM((NBUF, SPLIT, EMBED_DIM), jnp.float32),
        pltpu.SemaphoreType.DMA((NBUF,)),
        pltpu.SemaphoreType.DMA((NBUF,)),
    ],
    compiler_params=pltpu.CompilerParams(use_tc_tiling_on_sc=False),
)
def _gather_sc(x_hbm, table_hbm, out_hbm, idx_v, rows_v, gsem, ssem):
    wid = lax.axis_index("s") * NC + lax.axis_index("c")
    row0 = wid * BPW
    pltpu.sync_copy(x_hbm.at[pl.ds(row0, BPW)], idx_v)

    def chunk_coords(c, b_static):
        bl = c // 2
        if b_static % 2 == 0:
            return bl, 0, SPLIT
        return bl, SPLIT, SEQ - SPLIT

    def g_desc(c, b):
        bl, s0, ln = chunk_coords(c, b)
        return pltpu.make_async_copy(
            table_hbm.at[idx_v.at[bl, pl.ds(s0, ln)]],
            rows_v.at[b, pl.ds(0, ln)],
            gsem.at[b],
        )

    def s_desc(c, b):
        bl, s0, ln = chunk_coords(c, b)
        return pltpu.make_async_copy(
            rows_v.at[b, pl.ds(0, ln)],
            out_hbm.at[row0 + bl, pl.ds(s0, ln)],
            ssem.at[b],
        )

    for b in range(GLEAD + 1):
        g_desc(b, b).start()

    def outer(k, carry):
        c0 = k * NBUF
        for b in range(NBUF):
            c = c0 + b
            g_desc(c, b).wait()
            s_desc(c, b).start()
            j = c + GLEAD + 1
            bj = (b + GLEAD + 1) % NBUF

            @pl.when(jnp.logical_and(j >= NBUF, j < NCH))
            def _():
                s_desc(j - NBUF, bj).wait()

            @pl.when(j < NCH)
            def _():
                g_desc(j, bj).start()
        return carry

    lax.fori_loop(0, NCH // NBUF, outer, 0)

    for b in range(NBUF):
        c = NCH - NBUF + b
        s_desc(c, c % NBUF).wait()


def _mask_body(x_ref, m_ref):
    m_ref[...] = (x_ref[...] != PADDING_IDX).astype(jnp.float32)


def _mask_tc(x):
    return pl.pallas_call(
        _mask_body,
        out_shape=jax.ShapeDtypeStruct((BATCH, SEQ), jnp.float32),
        grid=(8,),
        in_specs=[pl.BlockSpec((BATCH // 8, SEQ), lambda i: (i, 0))],
        out_specs=pl.BlockSpec((BATCH // 8, SEQ), lambda i: (i, 0)),
    )(x)


def kernel(x, table):
    out = _gather_sc(x, table)
    mask = _mask_tc(x)
    return out, mask
